# baseline (speedup 1.0000x reference)
import jax
import jax.numpy as jnp
from jax import lax
from jax.experimental import pallas as pl
from jax.experimental.pallas import tpu as pltpu
from jax.experimental.pallas import tpu_sc as plsc

_B, _S = 4, 8192
_D = 768
_MAX_SEQ = 1024
_N = _B * _S

_NC, _NS = 2, 16
_NW = _NC * _NS
_CHUNK = _N // _NW
_WPR = _S // _CHUNK
_L = 16
_G = _CHUNK // _L
_SLOTS = 4
_K = 80


def _abacus_body(ids_hbm, table_hbm, out_hbm, ids_row, idx_v, stage, cache,
                 dirty, sem_out, sem_fb):
    cid = lax.axis_index("c")
    sid = lax.axis_index("s")
    w = sid * _NC + cid
    row = w // _WPR
    lbase = (w % _WPR) * _CHUNK

    pltpu.sync_copy(ids_hbm.at[pl.ds(pl.multiple_of(row * _S, _S), _S)],
                    ids_row)
    pltpu.sync_copy(table_hbm.at[pl.ds(0, _K)], cache)

    iota = lax.iota(jnp.int32, _L)
    ones = jnp.full((_L,), 1, jnp.int32)
    for s in range(_SLOTS):
        dirty[pl.ds(s * _L, _L)] = ones

    def prefix_body(i, vmax):
        off = pl.multiple_of(i * _L, _L)
        v = ids_row[pl.ds(off, _L)]
        dig = (v >= 4) & (v <= 13)
        t = jnp.where(dig, -1, i * _L + iota)
        return jnp.maximum(vmax, t)

    vmax0 = jnp.full((_L,), -1, jnp.int32)
    vmax = lax.fori_loop(0, lbase // _L, prefix_body, vmax0)
    carry0 = jnp.max(vmax)

    def scan_body(i, carry):
        off = lbase + i * _L
        v = ids_row[pl.ds(pl.multiple_of(off, _L), _L)]
        dig = (v >= 4) & (v <= 13)
        pos16 = off + iota
        t = jnp.where(dig, -1, pos16)
        m = jnp.maximum(plsc.cummax(t), carry)
        res = jnp.minimum((pos16 - m) * dig.astype(jnp.int32), _MAX_SEQ - 1)
        idx_v[pl.ds(pl.multiple_of(i * _L, _L), _L)] = res
        return jnp.max(m)

    lax.fori_loop(0, _CHUNK // _L, scan_body, carry0)

    obase = w * _CHUNK

    def group_body(g, _):
        slot = pl.multiple_of((g % _SLOTS) * _L, _L)

        @pl.when(g >= _SLOTS - 1)
        def _drain():
            pltpu.make_async_copy(out_hbm.at[pl.ds(0, _L)],
                                  stage.at[pl.ds(0, _L)], sem_out).wait()

        pv = idx_v[pl.ds(pl.multiple_of(g * _L, _L), _L)]
        dv = dirty[pl.ds(slot, _L)]
        need = ((pv > 2000000000) | (dv > 2000000000)).astype(jnp.int32)

        @pl.when(jnp.max(need) >= 0)
        def _build():
            enc = jnp.minimum(pv, _K - 1) + need * 65536
            es = [enc[l] for l in range(_L)]
            for l in range(_L):
                @pl.when(es[l] >= 65536)
                def _copy_row(e=es[l], l=l):
                    pc = e - 65536
                    for q in range(_D // (8 * _L)):
                        vals = [cache[pc, pl.ds((q * 8 + b) * _L, _L)]
                                for b in range(8)]
                        for b in range(8):
                            stage[slot + l,
                                  pl.ds((q * 8 + b) * _L, _L)] = vals[b]

        dirty[pl.ds(slot, _L)] = (pv > 0).astype(jnp.int32)

        @pl.when(jnp.max(pv) >= _K)
        def _fallback():
            pltpu.async_copy(
                table_hbm.at[idx_v.at[pl.ds(pl.multiple_of(g * _L, _L), _L)]],
                stage.at[pl.ds(slot, _L)], sem_fb).wait()
            dirty[pl.ds(slot, _L)] = ones

        pltpu.async_copy(
            stage.at[pl.ds(slot, _L)],
            out_hbm.at[pl.ds(pl.multiple_of(obase + g * _L, _L), _L)],
            sem_out)
        return 0

    lax.fori_loop(0, _G, group_body, 0)

    for _ in range(_SLOTS - 1):
        pltpu.make_async_copy(out_hbm.at[pl.ds(0, _L)],
                              stage.at[pl.ds(0, _L)], sem_out).wait()


@jax.jit
def kernel(input_ids, table):
    mesh = plsc.VectorSubcoreMesh(core_axis_name="c", subcore_axis_name="s")
    run = pl.kernel(
        _abacus_body,
        out_type=jax.ShapeDtypeStruct((_N, _D), jnp.float32),
        mesh=mesh,
        scratch_types=[
            pltpu.VMEM((_S,), jnp.int32),
            pltpu.VMEM((_CHUNK,), jnp.int32),
            pltpu.VMEM((_SLOTS * _L, _D), jnp.float32),
            pltpu.VMEM((_K, _D), jnp.float32),
            pltpu.VMEM((_SLOTS * _L,), jnp.int32),
            pltpu.SemaphoreType.DMA,
            pltpu.SemaphoreType.DMA,
        ],
        compiler_params=pltpu.CompilerParams(needs_layout_passes=False),
    )
    out = run(input_ids.reshape(-1), table)
    return out.reshape(_B, _S, _D)


# speedup vs baseline: 1.4780x; 1.4780x over previous
import jax
import jax.numpy as jnp
from jax import lax
from jax.experimental import pallas as pl
from jax.experimental.pallas import tpu as pltpu
from jax.experimental.pallas import tpu_sc as plsc

_B, _S = 4, 8192
_D = 768
_MAX_SEQ = 1024
_N = _B * _S

_NC, _NS = 2, 16
_NW = _NC * _NS
_CHUNK = _N // _NW
_WPR = _S // _CHUNK
_L = 16
_G = _CHUNK // _L
_SLOTS = 4


def _abacus_body(ids_hbm, table_hbm, out_hbm, ids_row, idx_v, stage, dirty,
                 table_sh, sem_out, sem_cp):
    cid = lax.axis_index("c")
    sid = lax.axis_index("s")
    w = sid * _NC + cid
    row = w // _WPR
    lbase = (w % _WPR) * _CHUNK

    pltpu.sync_copy(ids_hbm.at[pl.ds(pl.multiple_of(row * _S, _S), _S)],
                    ids_row)
    trows = _MAX_SEQ // _NS
    tbase = pl.multiple_of(sid * trows, trows)
    pltpu.sync_copy(table_hbm.at[pl.ds(tbase, trows)],
                    table_sh.at[pl.ds(tbase, trows)])

    iota = lax.iota(jnp.int32, _L)
    ones = jnp.full((_L,), 1, jnp.int32)
    for s in range(_SLOTS):
        dirty[pl.ds(s * _L, _L)] = ones

    def prefix_body(i, vmax):
        off = pl.multiple_of(i * _L, _L)
        v = ids_row[pl.ds(off, _L)]
        dig = (v >= 4) & (v <= 13)
        t = jnp.where(dig, -1, i * _L + iota)
        return jnp.maximum(vmax, t)

    vmax0 = jnp.full((_L,), -1, jnp.int32)
    vmax = lax.fori_loop(0, lbase // _L, prefix_body, vmax0)
    carry0 = jnp.max(vmax)

    def scan_body(i, carry):
        off = lbase + i * _L
        v = ids_row[pl.ds(pl.multiple_of(off, _L), _L)]
        dig = (v >= 4) & (v <= 13)
        pos16 = off + iota
        t = jnp.where(dig, -1, pos16)
        m = jnp.maximum(plsc.cummax(t), carry)
        res = jnp.minimum((pos16 - m) * dig.astype(jnp.int32), _MAX_SEQ - 1)
        idx_v[pl.ds(pl.multiple_of(i * _L, _L), _L)] = res
        return jnp.max(m)

    lax.fori_loop(0, _CHUNK // _L, scan_body, carry0)

    plsc.subcore_barrier()

    obase = w * _CHUNK

    def group_body(g, _):
        slot = pl.multiple_of((g % _SLOTS) * _L, _L)

        @pl.when(g >= _SLOTS - 1)
        def _drain():
            pltpu.make_async_copy(out_hbm.at[pl.ds(0, _L)],
                                  stage.at[pl.ds(0, _L)], sem_out).wait()

        pv = idx_v[pl.ds(pl.multiple_of(g * _L, _L), _L)]
        dv = dirty[pl.ds(slot, _L)]
        need = ((pv > 0) | (dv > 0)).astype(jnp.int32)

        @pl.when(jnp.max(need) > 0)
        def _build():
            enc = pv + need * 65536
            es = [enc[l] for l in range(_L)]
            for l in range(_L):
                @pl.when(es[l] >= 65536)
                def _copy_row(e=es[l], l=l):
                    pltpu.async_copy(table_sh.at[e - 65536],
                                     stage.at[slot + l], sem_cp)
            for l in range(_L):
                @pl.when(es[l] >= 65536)
                def _wait_row():
                    pltpu.make_async_copy(table_hbm.at[0], stage.at[0],
                                          sem_cp).wait()

        dirty[pl.ds(slot, _L)] = (pv > 0).astype(jnp.int32)

        pltpu.async_copy(
            stage.at[pl.ds(slot, _L)],
            out_hbm.at[pl.ds(pl.multiple_of(obase + g * _L, _L), _L)],
            sem_out)
        return 0

    lax.fori_loop(0, _G, group_body, 0)

    for _ in range(_SLOTS - 1):
        pltpu.make_async_copy(out_hbm.at[pl.ds(0, _L)],
                              stage.at[pl.ds(0, _L)], sem_out).wait()


@jax.jit
def kernel(input_ids, table):
    mesh = plsc.VectorSubcoreMesh(core_axis_name="c", subcore_axis_name="s")
    run = pl.kernel(
        _abacus_body,
        out_type=jax.ShapeDtypeStruct((_N, _D), jnp.float32),
        mesh=mesh,
        scratch_types=[
            pltpu.VMEM((_S,), jnp.int32),
            pltpu.VMEM((_CHUNK,), jnp.int32),
            pltpu.VMEM((_SLOTS * _L, _D), jnp.float32),
            pltpu.VMEM((_SLOTS * _L,), jnp.int32),
            pltpu.VMEM_SHARED((_MAX_SEQ, _D), jnp.float32),
            pltpu.SemaphoreType.DMA,
            pltpu.SemaphoreType.DMA,
        ],
        compiler_params=pltpu.CompilerParams(needs_layout_passes=False),
    )
    out = run(input_ids.reshape(-1), table)
    return out.reshape(_B, _S, _D)


# speedup vs baseline: 1.5305x; 1.0355x over previous
import jax
import jax.numpy as jnp
from jax import lax
from jax.experimental import pallas as pl
from jax.experimental.pallas import tpu as pltpu
from jax.experimental.pallas import tpu_sc as plsc

_B, _S = 4, 8192
_D = 768
_MAX_SEQ = 1024
_N = _B * _S

_NC, _NS = 2, 16
_NW = _NC * _NS
_CHUNK = _N // _NW
_WPR = _S // _CHUNK
_L = 16
_G = _CHUNK // _L
_SLOTS = 4


def _abacus_body(ids_hbm, table_hbm, out_hbm, ids_row, idx_v, stage, dirty,
                 table_sh, sem_out, sem_cp):
    cid = lax.axis_index("c")
    sid = lax.axis_index("s")
    w = sid * _NC + cid
    row = w // _WPR
    lbase = (w % _WPR) * _CHUNK

    trows = _MAX_SEQ // _NS
    tbase = pl.multiple_of(sid * trows, trows)
    tstage = pltpu.async_copy(table_hbm.at[pl.ds(tbase, trows)],
                              table_sh.at[pl.ds(tbase, trows)], sem_cp)
    pltpu.sync_copy(ids_hbm.at[pl.ds(pl.multiple_of(row * _S, _S), _S)],
                    ids_row)

    iota = lax.iota(jnp.int32, _L)
    ones = jnp.full((_L,), 1, jnp.int32)
    for s in range(_SLOTS):
        dirty[pl.ds(s * _L, _L)] = ones

    def prefix_body(i, vmax):
        off = pl.multiple_of(i * _L, _L)
        v = ids_row[pl.ds(off, _L)]
        dig = (v >= 4) & (v <= 13)
        t = jnp.where(dig, -1, i * _L + iota)
        return jnp.maximum(vmax, t)

    vmax0 = jnp.full((_L,), -1, jnp.int32)
    vmax = lax.fori_loop(0, lbase // _L, prefix_body, vmax0)
    carry0 = jnp.max(vmax)

    def scan_body(i, carry):
        off = lbase + i * _L
        v = ids_row[pl.ds(pl.multiple_of(off, _L), _L)]
        dig = (v >= 4) & (v <= 13)
        pos16 = off + iota
        t = jnp.where(dig, -1, pos16)
        m = jnp.maximum(plsc.cummax(t), carry)
        res = jnp.minimum((pos16 - m) * dig.astype(jnp.int32), _MAX_SEQ - 1)
        idx_v[pl.ds(pl.multiple_of(i * _L, _L), _L)] = res
        return jnp.max(m)

    lax.fori_loop(0, _CHUNK // _L, scan_body, carry0)

    tstage.wait()
    plsc.subcore_barrier()

    obase = w * _CHUNK

    def group_body(g, _):
        slot = pl.multiple_of((g % _SLOTS) * _L, _L)

        @pl.when((g % 2 == 0) & (g >= _SLOTS))
        def _drain():
            pltpu.make_async_copy(out_hbm.at[pl.ds(0, 2 * _L)],
                                  stage.at[pl.ds(0, 2 * _L)], sem_out).wait()

        pv = idx_v[pl.ds(pl.multiple_of(g * _L, _L), _L)]
        dv = dirty[pl.ds(slot, _L)]
        need = ((pv > 0) | (dv > 0)).astype(jnp.int32)

        @pl.when(jnp.max(need) > 0)
        def _build():
            enc = pv + need * 65536
            es = [enc[l] for l in range(_L)]
            for l in range(_L):
                @pl.when(es[l] >= 65536)
                def _copy_row(e=es[l], l=l):
                    pltpu.async_copy(table_sh.at[e - 65536],
                                     stage.at[slot + l], sem_cp)
            for l in range(_L):
                @pl.when(es[l] >= 65536)
                def _wait_row():
                    pltpu.make_async_copy(table_hbm.at[0], stage.at[0],
                                          sem_cp).wait()

        dirty[pl.ds(slot, _L)] = (pv > 0).astype(jnp.int32)

        @pl.when(g % 2 == 1)
        def _scatter():
            half = pl.multiple_of(((g // 2) % 2) * 2 * _L, 2 * _L)
            pltpu.async_copy(
                stage.at[pl.ds(half, 2 * _L)],
                out_hbm.at[pl.ds(
                    pl.multiple_of(obase + (g - 1) * _L, 2 * _L), 2 * _L)],
                sem_out)
        return 0

    lax.fori_loop(0, _G, group_body, 0)

    for _ in range(2):
        pltpu.make_async_copy(out_hbm.at[pl.ds(0, 2 * _L)],
                              stage.at[pl.ds(0, 2 * _L)], sem_out).wait()


@jax.jit
def kernel(input_ids, table):
    mesh = plsc.VectorSubcoreMesh(core_axis_name="c", subcore_axis_name="s")
    run = pl.kernel(
        _abacus_body,
        out_type=jax.ShapeDtypeStruct((_N, _D), jnp.float32),
        mesh=mesh,
        scratch_types=[
            pltpu.VMEM((_S,), jnp.int32),
            pltpu.VMEM((_CHUNK,), jnp.int32),
            pltpu.VMEM((_SLOTS * _L, _D), jnp.float32),
            pltpu.VMEM((_SLOTS * _L,), jnp.int32),
            pltpu.VMEM_SHARED((_MAX_SEQ, _D), jnp.float32),
            pltpu.SemaphoreType.DMA,
            pltpu.SemaphoreType.DMA,
        ],
        compiler_params=pltpu.CompilerParams(needs_layout_passes=False),
    )
    out = run(input_ids.reshape(-1), table)
    return out.reshape(_B, _S, _D)


# speedup vs baseline: 1.5523x; 1.0142x over previous
import jax
import jax.numpy as jnp
from jax import lax
from jax.experimental import pallas as pl
from jax.experimental.pallas import tpu as pltpu
from jax.experimental.pallas import tpu_sc as plsc

_B, _S = 4, 8192
_D = 768
_MAX_SEQ = 1024
_N = _B * _S

_NC, _NS = 2, 16
_NW = _NC * _NS
_CHUNK = _N // _NW
_WPR = _S // _CHUNK
_L = 16
_G = _CHUNK // _L
_SLOTS = 4


def _abacus_body(ids_hbm, table_hbm, out_hbm, ids_row, idx_v, stage, dirty,
                 table_sh, sem_out, sem_cp):
    cid = lax.axis_index("c")
    sid = lax.axis_index("s")
    w = sid * _NC + cid
    row = w // _WPR
    lbase = (w % _WPR) * _CHUNK

    trows = _MAX_SEQ // _NS
    tbase = pl.multiple_of(sid * trows, trows)
    tstage = pltpu.async_copy(table_hbm.at[pl.ds(tbase, trows)],
                              table_sh.at[pl.ds(tbase, trows)], sem_cp)
    pltpu.sync_copy(ids_hbm.at[pl.ds(pl.multiple_of(row * _S, _S), _S)],
                    ids_row)

    iota = lax.iota(jnp.int32, _L)
    ones = jnp.full((_L,), 1, jnp.int32)
    for s in range(_SLOTS):
        dirty[pl.ds(s * _L, _L)] = ones

    def prefix_body(i, vmax):
        ts = []
        for u in range(4):
            off = pl.multiple_of(i * 4 * _L + u * _L, _L)
            v = ids_row[pl.ds(off, _L)]
            dig = (v >= 4) & (v <= 13)
            ts.append(jnp.where(dig, -1, i * 4 * _L + u * _L + iota))
        return jnp.maximum(vmax,
                           jnp.maximum(jnp.maximum(ts[0], ts[1]),
                                       jnp.maximum(ts[2], ts[3])))

    vmax0 = jnp.full((_L,), -1, jnp.int32)
    vmax = lax.fori_loop(0, lbase // (4 * _L), prefix_body, vmax0)
    carry0 = jnp.max(vmax)

    def scan_body(i, carry):
        off = lbase + i * _L
        v = ids_row[pl.ds(pl.multiple_of(off, _L), _L)]
        dig = (v >= 4) & (v <= 13)
        pos16 = off + iota
        t = jnp.where(dig, -1, pos16)
        m = jnp.maximum(plsc.cummax(t), carry)
        res = jnp.minimum((pos16 - m) * dig.astype(jnp.int32), _MAX_SEQ - 1)
        idx_v[pl.ds(pl.multiple_of(i * _L, _L), _L)] = res
        return jnp.max(m)

    lax.fori_loop(0, _CHUNK // _L, scan_body, carry0)

    tstage.wait()
    plsc.subcore_barrier()

    obase = w * _CHUNK

    def group_body(g, _):
        slot = pl.multiple_of((g % _SLOTS) * _L, _L)

        @pl.when((g % 2 == 0) & (g >= _SLOTS))
        def _drain():
            pltpu.make_async_copy(out_hbm.at[pl.ds(0, 2 * _L)],
                                  stage.at[pl.ds(0, 2 * _L)], sem_out).wait()

        pv = idx_v[pl.ds(pl.multiple_of(g * _L, _L), _L)]
        dv = dirty[pl.ds(slot, _L)]
        need = ((pv > 0) | (dv > 0)).astype(jnp.int32)

        @pl.when(jnp.max(need) > 0)
        def _build():
            enc = pv + need * 65536
            es = [enc[l] for l in range(_L)]
            for l in range(_L):
                @pl.when(es[l] >= 65536)
                def _copy_row(e=es[l], l=l):
                    pltpu.async_copy(table_sh.at[e - 65536],
                                     stage.at[slot + l], sem_cp)
            for l in range(_L):
                @pl.when(es[l] >= 65536)
                def _wait_row():
                    pltpu.make_async_copy(table_hbm.at[0], stage.at[0],
                                          sem_cp).wait()

        dirty[pl.ds(slot, _L)] = (pv > 0).astype(jnp.int32)

        @pl.when(g % 2 == 1)
        def _scatter():
            half = pl.multiple_of(((g // 2) % 2) * 2 * _L, 2 * _L)
            pltpu.async_copy(
                stage.at[pl.ds(half, 2 * _L)],
                out_hbm.at[pl.ds(
                    pl.multiple_of(obase + (g - 1) * _L, 2 * _L), 2 * _L)],
                sem_out)
        return 0

    lax.fori_loop(0, _G, group_body, 0)

    for _ in range(2):
        pltpu.make_async_copy(out_hbm.at[pl.ds(0, 2 * _L)],
                              stage.at[pl.ds(0, 2 * _L)], sem_out).wait()


@jax.jit
def kernel(input_ids, table):
    mesh = plsc.VectorSubcoreMesh(core_axis_name="c", subcore_axis_name="s")
    run = pl.kernel(
        _abacus_body,
        out_type=jax.ShapeDtypeStruct((_N, _D), jnp.float32),
        mesh=mesh,
        scratch_types=[
            pltpu.VMEM((_S,), jnp.int32),
            pltpu.VMEM((_CHUNK,), jnp.int32),
            pltpu.VMEM((_SLOTS * _L, _D), jnp.float32),
            pltpu.VMEM((_SLOTS * _L,), jnp.int32),
            pltpu.VMEM_SHARED((_MAX_SEQ, _D), jnp.float32),
            pltpu.SemaphoreType.DMA,
            pltpu.SemaphoreType.DMA,
        ],
        compiler_params=pltpu.CompilerParams(needs_layout_passes=False),
    )
    out = run(input_ids.reshape(-1), table)
    return out.reshape(_B, _S, _D)


# speedup vs baseline: 1.5612x; 1.0057x over previous
import jax
import jax.numpy as jnp
from jax import lax
from jax.experimental import pallas as pl
from jax.experimental.pallas import tpu as pltpu
from jax.experimental.pallas import tpu_sc as plsc

_B, _S = 4, 8192
_D = 768
_MAX_SEQ = 1024
_N = _B * _S

_NC, _NS = 2, 16
_NW = _NC * _NS
_CHUNK = _N // _NW
_WPR = _S // _CHUNK
_L = 16
_G = _CHUNK // _L
_SLOTS = 4


def _abacus_body(ids_hbm, table_hbm, out_hbm, ids_row, idx_v, stage, dirty,
                 table_sh, sem_out, sem_cp):
    cid = lax.axis_index("c")
    sid = lax.axis_index("s")
    w = sid * _NC + cid
    row = w // _WPR
    lbase = (w % _WPR) * _CHUNK

    trows = _MAX_SEQ // _NS
    tbase = pl.multiple_of(sid * trows, trows)
    tstage = pltpu.async_copy(table_hbm.at[pl.ds(tbase, trows)],
                              table_sh.at[pl.ds(tbase, trows)], sem_cp)
    pltpu.sync_copy(ids_hbm.at[pl.ds(pl.multiple_of(row * _S, _S), _S)],
                    ids_row)

    iota = lax.iota(jnp.int32, _L)
    ones = jnp.full((_L,), 1, jnp.int32)
    for s in range(_SLOTS):
        dirty[pl.ds(s * _L, _L)] = ones

    def prefix_body(i, vmax):
        ts = []
        for u in range(4):
            off = pl.multiple_of(i * 4 * _L + u * _L, _L)
            v = ids_row[pl.ds(off, _L)]
            dig = (v >= 4) & (v <= 13)
            ts.append(jnp.where(dig, -1, i * 4 * _L + u * _L + iota))
        return jnp.maximum(vmax,
                           jnp.maximum(jnp.maximum(ts[0], ts[1]),
                                       jnp.maximum(ts[2], ts[3])))

    vmax0 = jnp.full((_L,), -1, jnp.int32)
    vmax = lax.fori_loop(0, lbase // (4 * _L), prefix_body, vmax0)
    carry0 = jnp.max(vmax)

    def scan_body(i, carry):
        off = lbase + i * _L
        v = ids_row[pl.ds(pl.multiple_of(off, _L), _L)]
        dig = (v >= 4) & (v <= 13)
        pos16 = off + iota
        t = jnp.where(dig, -1, pos16)
        m = jnp.maximum(plsc.cummax(t), carry)
        res = jnp.minimum((pos16 - m) * dig.astype(jnp.int32), _MAX_SEQ - 1)
        idx_v[pl.ds(pl.multiple_of(i * _L, _L), _L)] = res
        return jnp.max(m)

    lax.fori_loop(0, _CHUNK // _L, scan_body, carry0)

    tstage.wait()
    plsc.subcore_barrier()

    obase = w * _CHUNK

    def group_body(g, pending):
        slot = pl.multiple_of((g % _SLOTS) * _L, _L)

        @pl.when((g % 2 == 0) & (g >= _SLOTS))
        def _drain():
            pltpu.make_async_copy(out_hbm.at[pl.ds(0, 2 * _L)],
                                  stage.at[pl.ds(0, 2 * _L)], sem_out).wait()

        pv = idx_v[pl.ds(pl.multiple_of(g * _L, _L), _L)]
        dv = dirty[pl.ds(slot, _L)]
        need = ((pv > 0) | (dv > 0)).astype(jnp.int32)

        @pl.when(jnp.max(need) > 0)
        def _build():
            enc = pv + need * 65536
            es = [enc[l] for l in range(_L)]
            for l in range(_L):
                @pl.when(es[l] >= 65536)
                def _copy_row(e=es[l], l=l):
                    pltpu.async_copy(table_sh.at[e - 65536],
                                     stage.at[slot + l], sem_cp)

        dirty[pl.ds(slot, _L)] = (pv > 0).astype(jnp.int32)

        total = pending + jnp.sum(need)
        wait_n = jnp.where(g % 2 == 1, total, 0)

        def wait_body(i, c):
            pltpu.make_async_copy(table_hbm.at[0], stage.at[0],
                                  sem_cp).wait()
            return c

        lax.fori_loop(0, wait_n, wait_body, 0)

        @pl.when(g % 2 == 1)
        def _scatter():
            half = pl.multiple_of(((g // 2) % 2) * 2 * _L, 2 * _L)
            pltpu.async_copy(
                stage.at[pl.ds(half, 2 * _L)],
                out_hbm.at[pl.ds(
                    pl.multiple_of(obase + (g - 1) * _L, 2 * _L), 2 * _L)],
                sem_out)

        return jnp.where(g % 2 == 1, 0, total)

    lax.fori_loop(0, _G, group_body, jnp.int32(0))

    for _ in range(2):
        pltpu.make_async_copy(out_hbm.at[pl.ds(0, 2 * _L)],
                              stage.at[pl.ds(0, 2 * _L)], sem_out).wait()


@jax.jit
def kernel(input_ids, table):
    mesh = plsc.VectorSubcoreMesh(core_axis_name="c", subcore_axis_name="s")
    run = pl.kernel(
        _abacus_body,
        out_type=jax.ShapeDtypeStruct((_N, _D), jnp.float32),
        mesh=mesh,
        scratch_types=[
            pltpu.VMEM((_S,), jnp.int32),
            pltpu.VMEM((_CHUNK,), jnp.int32),
            pltpu.VMEM((_SLOTS * _L, _D), jnp.float32),
            pltpu.VMEM((_SLOTS * _L,), jnp.int32),
            pltpu.VMEM_SHARED((_MAX_SEQ, _D), jnp.float32),
            pltpu.SemaphoreType.DMA,
            pltpu.SemaphoreType.DMA,
        ],
        compiler_params=pltpu.CompilerParams(needs_layout_passes=False),
    )
    out = run(input_ids.reshape(-1), table)
    return out.reshape(_B, _S, _D)
